# Initial kernel scaffold; baseline (speedup 1.0000x reference)
import jax
import jax.numpy as jnp
from jax.experimental import pallas as pl


def kernel(features, edge_index, W1, b1, W2, b2, W3, b3):
    raise NotImplementedError("write your pallas kernel here")


# speedup vs baseline: 2.5345x; 2.5345x over previous
import functools

import jax
import jax.numpy as jnp
from jax import lax
from jax.experimental import pallas as pl
from jax.experimental.pallas import tpu as pltpu
from jax.experimental.pallas import tpu_sc as plsc

N = 5000
IN_DIM = 1000
NHID = 256
OUT_DIM = 2

NC = 2
NS = 16
NW = NC * NS
BATCH = 128
NSLAB = 8
EPW = BATCH * NSLAB
EPAD = EPW * NW
TRASH_ROW = N
ACC_ROWS = 5120
RPT = ACC_ROWS // NS
TAIL = N - (NS - 1) * RPT

@functools.lru_cache(maxsize=None)
def _mesh():
    return plsc.VectorSubcoreMesh(
        core_axis_name="c", subcore_axis_name="s",
        num_cores=NC, num_subcores=NS)


def _const_fill(buf, rows, d, val):
    def body(i, _):
        row = i // (d // 16)
        col = (i % (d // 16)) * 16
        buf[row, pl.ds(col, 16)] = jnp.full((16,), val, jnp.float32)
        return ()
    lax.fori_loop(0, rows * (d // 16), body, ())


def _zero_fill(zbuf, zrows, d):
    _const_fill(zbuf, zrows, d, 0.0)


def _zero_acc(acc, zbuf, zrows, sid):
    base = sid * RPT
    nfull = RPT // zrows
    rem = RPT - nfull * zrows
    for k in range(nfull):
        pltpu.sync_copy(zbuf, acc.at[pl.ds(base + k * zrows, zrows)])
    if rem:
        pltpu.sync_copy(zbuf.at[pl.ds(0, rem)],
                        acc.at[pl.ds(base + nfull * zrows, rem)])


def _write_back(acc, out, cid, sid):
    base = sid * RPT
    pltpu.sync_copy(acc.at[pl.ds(base, TAIL)],
                    out.at[cid, pl.ds(base, TAIL)])

    @pl.when(sid < NS - 1)
    def _():
        pltpu.sync_copy(acc.at[pl.ds(base + TAIL, RPT - TAIL)],
                        out.at[cid, pl.ds(base + TAIL, RPT - TAIL)])


DH = NHID // NC
NSLAB_W = EPAD // (NS * BATCH)


@functools.lru_cache(maxsize=None)
def _make_sc_scatter_wide():
    @functools.partial(
        pl.kernel,
        out_type=jax.ShapeDtypeStruct((NC, N, DH), jnp.float32),
        mesh=_mesh(),
        scratch_types=[
            pltpu.VMEM((NSLAB_W, BATCH), jnp.int32),
            pltpu.VMEM((NSLAB_W, BATCH), jnp.int32),
            pltpu.VMEM((BATCH, DH), jnp.float32),
            pltpu.VMEM((BATCH, DH), jnp.float32),
            pltpu.VMEM((64, DH), jnp.float32),
            pltpu.VMEM_SHARED((ACC_ROWS, DH), jnp.float32),
            pltpu.SemaphoreType.DMA,
            pltpu.SemaphoreType.DMA,
        ],
    )
    def sc_scatter(table, src_idx, dst_idx, out,
                   src_s, dst_s, rowa, rowb, zbuf, acc, sema, semb):
        cid = lax.axis_index("c")
        sid = lax.axis_index("s")
        pltpu.sync_copy(src_idx.at[sid], src_s)
        pltpu.sync_copy(dst_idx.at[sid], dst_s)
        _zero_fill(zbuf, 64, DH)
        _zero_acc(acc, zbuf, 64, sid)
        plsc.subcore_barrier()

        half = table.at[cid]
        bufs = (rowa, rowb)
        sems = (sema, semb)
        descs = [None, None]
        descs[0] = pltpu.async_copy(half.at[src_s.at[0]], bufs[0], sems[0])
        for j in range(NSLAB_W):
            if j + 1 < NSLAB_W:
                descs[(j + 1) % 2] = pltpu.async_copy(
                    half.at[src_s.at[j + 1]], bufs[(j + 1) % 2],
                    sems[(j + 1) % 2])
            descs[j % 2].wait()
            pltpu.sync_copy(bufs[j % 2], acc.at[dst_s.at[j]], add=True)

        plsc.subcore_barrier()
        _write_back(acc, out, cid, sid)

    return sc_scatter


def _sc_scatter_wide(table, src_p, dst_p):
    return _make_sc_scatter_wide()(table, src_p, dst_p)


@functools.lru_cache(maxsize=None)
def _make_sc_deg():
    @functools.partial(
        pl.kernel,
        out_type=jax.ShapeDtypeStruct((NC, N, DH), jnp.float32),
        mesh=_mesh(),
        scratch_types=[
            pltpu.VMEM((NSLAB, BATCH), jnp.int32),
            pltpu.VMEM((BATCH, DH), jnp.float32),
            pltpu.VMEM((64, DH), jnp.float32),
            pltpu.VMEM_SHARED((ACC_ROWS, DH), jnp.float32),
        ],
    )
    def sc_deg(dst_idx, out, dst_s, rowbuf, zbuf, acc):
        cid = lax.axis_index("c")
        sid = lax.axis_index("s")
        wid = cid * NS + sid
        pltpu.sync_copy(dst_idx.at[wid], dst_s)
        _const_fill(rowbuf, BATCH, DH, 1.0)
        _zero_fill(zbuf, 64, DH)
        _zero_acc(acc, zbuf, 64, sid)
        plsc.subcore_barrier()
        for j in range(NSLAB):
            pltpu.sync_copy(rowbuf, acc.at[dst_s.at[j]], add=True)
        plsc.subcore_barrier()
        _write_back(acc, out, cid, sid)

    return sc_deg


def _sc_deg(dst_p):
    return _make_sc_deg()(dst_p)


def _dinv_from_deg(deg_blk):
    deg = deg_blk[0, :, 0:1] + deg_blk[1, :, 0:1] + 1.0
    return lax.rsqrt(deg)


def _k1_body(feat_ref, w1_ref, deg_ref, out_ref):
    dinv = _dinv_from_deg(deg_ref[...])
    h = jnp.dot(feat_ref[...], w1_ref[...],
                preferred_element_type=jnp.float32) * dinv
    out_ref[0] = h[:, :DH]
    out_ref[1] = h[:, DH:]


def _k_combine_body(agg_ref, hs_ref, deg_ref, b_ref, w_ref, out_ref):
    dinv = _dinv_from_deg(deg_ref[...])
    s = jnp.concatenate(
        [agg_ref[0] + hs_ref[0], agg_ref[1] + hs_ref[1]], axis=-1)
    t = jnp.maximum(s * dinv + b_ref[...], 0.0)
    r = jnp.dot(t, w_ref[...], preferred_element_type=jnp.float32) * dinv
    out_ref[0] = r[:, :DH]
    out_ref[1] = r[:, DH:]


def _k_combine3_body(agg_ref, hs_ref, deg_ref, b_ref, out_ref):
    dinv = _dinv_from_deg(deg_ref[...])
    s = jnp.concatenate(
        [agg_ref[0] + hs_ref[0], agg_ref[1] + hs_ref[1]], axis=-1)
    t = jnp.maximum(s * dinv + b_ref[...], 0.0) * dinv
    out_ref[0] = t[:, :DH]
    out_ref[1] = t[:, DH:]


def _k_emb_body(agg_ref, hs_ref, deg_ref, w_ref, b_ref, out_ref):
    dinv = _dinv_from_deg(deg_ref[...])
    u = jnp.concatenate(
        [agg_ref[0] + hs_ref[0], agg_ref[1] + hs_ref[1]], axis=-1) * dinv
    out_ref[...] = jnp.dot(u, w_ref[...],
                           preferred_element_type=jnp.float32) + b_ref[...]

def _k_q_body(emb_ref, embt_ref, out_ref):
    xi = emb_ref[:, 0:1]
    yi = emb_ref[:, 1:2]
    xj = embt_ref[0:1, :]
    yj = embt_ref[1:2, :]
    dx = xi - xj
    dy = yi - yj
    dist = jnp.sqrt(dx * dx + dy * dy)
    out_ref[...] = 1.0 / (1.0 + 0.5 * dist)


BM = 1000
QBM = 200


def kernel(features, edge_index, W1, b1, W2, b2, W3, b3):
    src = edge_index[0]
    dst = edge_index[1]
    e = src.shape[0]
    src_flat = jnp.concatenate([src, jnp.zeros((EPAD - e,), jnp.int32)])
    dst_flat = jnp.concatenate(
        [dst, jnp.full((EPAD - e,), TRASH_ROW, jnp.int32)])
    src_n = src_flat.reshape(NW, NSLAB, BATCH)
    dst_n = dst_flat.reshape(NW, NSLAB, BATCH)
    src_w = src_flat.reshape(NS, NSLAB_W, BATCH)
    dst_w = dst_flat.reshape(NS, NSLAB_W, BATCH)
    b1r = b1.reshape(1, NHID)
    b2r = b2.reshape(1, NHID)
    b3r = b3.reshape(1, OUT_DIM)

    degpart = _sc_deg(dst_n)

    grid = N // BM
    h1s = pl.pallas_call(
        _k1_body,
        grid=(grid,),
        in_specs=[
            pl.BlockSpec((BM, IN_DIM), lambda i: (i, 0)),
            pl.BlockSpec((IN_DIM, NHID), lambda i: (0, 0)),
            pl.BlockSpec((NC, BM, DH), lambda i: (0, i, 0)),
        ],
        out_specs=pl.BlockSpec((NC, BM, DH), lambda i: (0, i, 0)),
        out_shape=jax.ShapeDtypeStruct((NC, N, DH), jnp.float32),
    )(features, W1, degpart)

    def combine(body, agg, hs, b, w, out_shape, out_spec):
        return pl.pallas_call(
            body,
            grid=(grid,),
            in_specs=[
                pl.BlockSpec((NC, BM, DH), lambda i: (0, i, 0)),
                pl.BlockSpec((NC, BM, DH), lambda i: (0, i, 0)),
                pl.BlockSpec((NC, BM, DH), lambda i: (0, i, 0)),
                pl.BlockSpec((1, NHID), lambda i: (0, 0)),
                pl.BlockSpec((NHID, w.shape[1]), lambda i: (0, 0)),
            ],
            out_specs=out_spec,
            out_shape=out_shape,
        )(agg, hs, degpart, b, w)

    wide_shape = jax.ShapeDtypeStruct((NC, N, DH), jnp.float32)
    wide_spec = pl.BlockSpec((NC, BM, DH), lambda i: (0, i, 0))
    agg1 = _sc_scatter_wide(h1s, src_w, dst_w)
    h2s = combine(_k_combine_body, agg1, h1s, b1r, W2, wide_shape, wide_spec)
    agg2 = _sc_scatter_wide(h2s, src_w, dst_w)
    t3s = pl.pallas_call(
        _k_combine3_body,
        grid=(grid,),
        in_specs=[
            pl.BlockSpec((NC, BM, DH), lambda i: (0, i, 0)),
            pl.BlockSpec((NC, BM, DH), lambda i: (0, i, 0)),
            pl.BlockSpec((NC, BM, DH), lambda i: (0, i, 0)),
            pl.BlockSpec((1, NHID), lambda i: (0, 0)),
        ],
        out_specs=wide_spec,
        out_shape=wide_shape,
    )(agg2, h2s, degpart, b2r)
    agg3 = _sc_scatter_wide(t3s, src_w, dst_w)

    emb = pl.pallas_call(
        _k_emb_body,
        grid=(grid,),
        in_specs=[
            pl.BlockSpec((NC, BM, DH), lambda i: (0, i, 0)),
            pl.BlockSpec((NC, BM, DH), lambda i: (0, i, 0)),
            pl.BlockSpec((NC, BM, DH), lambda i: (0, i, 0)),
            pl.BlockSpec((NHID, OUT_DIM), lambda i: (0, 0)),
            pl.BlockSpec((1, OUT_DIM), lambda i: (0, 0)),
        ],
        out_specs=pl.BlockSpec((BM, OUT_DIM), lambda i: (i, 0)),
        out_shape=jax.ShapeDtypeStruct((N, OUT_DIM), jnp.float32),
    )(agg3, t3s, degpart, W3, b3r)

    q = pl.pallas_call(
        _k_q_body,
        grid=(N // QBM,),
        in_specs=[
            pl.BlockSpec((QBM, OUT_DIM), lambda i: (i, 0)),
            pl.BlockSpec((OUT_DIM, N), lambda i: (0, 0)),
        ],
        out_specs=pl.BlockSpec((QBM, N), lambda i: (i, 0)),
        out_shape=jax.ShapeDtypeStruct((N, N), jnp.float32),
    )(emb, emb.T)

    return emb, q


# speedup vs baseline: 5.0664x; 1.9989x over previous
import functools

import jax
import jax.numpy as jnp
from jax import lax
from jax.experimental import pallas as pl
from jax.experimental.pallas import tpu as pltpu
from jax.experimental.pallas import tpu_sc as plsc

N = 5000
IN_DIM = 1000
NHID = 256
OUT_DIM = 2

NC = 2
NS = 16
NW = NC * NS
BATCH = 128
NSLAB = 8
EPW = BATCH * NSLAB
EPAD = EPW * NW
TRASH_ROW = N
ACC_ROWS = 5120
RPT = ACC_ROWS // NS
TAIL = N - (NS - 1) * RPT

@functools.lru_cache(maxsize=None)
def _mesh():
    return plsc.VectorSubcoreMesh(
        core_axis_name="c", subcore_axis_name="s",
        num_cores=NC, num_subcores=NS)


def _const_fill(buf, rows, d, val):
    def body(i, _):
        row = i // (d // 16)
        col = (i % (d // 16)) * 16
        buf[row, pl.ds(col, 16)] = jnp.full((16,), val, jnp.float32)
        return ()
    lax.fori_loop(0, rows * (d // 16), body, ())


def _zero_fill(zbuf, zrows, d):
    _const_fill(zbuf, zrows, d, 0.0)


def _zero_acc(acc, zbuf, zrows, sid):
    base = sid * RPT
    nfull = RPT // zrows
    rem = RPT - nfull * zrows
    for k in range(nfull):
        pltpu.sync_copy(zbuf, acc.at[pl.ds(base + k * zrows, zrows)])
    if rem:
        pltpu.sync_copy(zbuf.at[pl.ds(0, rem)],
                        acc.at[pl.ds(base + nfull * zrows, rem)])


def _write_back(acc, out, cid, sid):
    base = sid * RPT
    pltpu.sync_copy(acc.at[pl.ds(base, TAIL)],
                    out.at[cid, pl.ds(base, TAIL)])

    @pl.when(sid < NS - 1)
    def _():
        pltpu.sync_copy(acc.at[pl.ds(base + TAIL, RPT - TAIL)],
                        out.at[cid, pl.ds(base + TAIL, RPT - TAIL)])


DH = NHID // NC
NSLAB_W = 15
WEPAD = NSLAB_W * BATCH * NS
NBUF = 5
LOOK = 2
ZROWS = 16


@functools.lru_cache(maxsize=None)
def _make_sc_scatter_wide():
    @functools.partial(
        pl.kernel,
        out_type=jax.ShapeDtypeStruct((NC, N, DH), jnp.float32),
        mesh=_mesh(),
        scratch_types=[
            pltpu.VMEM((NSLAB_W, BATCH), jnp.int32),
            pltpu.VMEM((NSLAB_W, BATCH), jnp.int32),
            [pltpu.VMEM((BATCH, DH), jnp.float32)] * NBUF,
            pltpu.VMEM((ZROWS, DH), jnp.float32),
            pltpu.VMEM_SHARED((ACC_ROWS, DH), jnp.float32),
            [pltpu.SemaphoreType.DMA] * NBUF,
            [pltpu.SemaphoreType.DMA] * NBUF,
        ],
    )
    def sc_scatter(table, src_idx, dst_idx, out,
                   src_s, dst_s, bufs, zbuf, acc, gsems, ssems):
        cid = lax.axis_index("c")
        sid = lax.axis_index("s")
        pltpu.sync_copy(src_idx.at[sid], src_s)
        pltpu.sync_copy(dst_idx.at[sid], dst_s)
        _zero_fill(zbuf, ZROWS, DH)
        _zero_acc(acc, zbuf, ZROWS, sid)
        plsc.subcore_barrier()

        half = table.at[cid]
        gdesc = [None] * NBUF
        sdesc = [None] * NBUF
        for j in range(LOOK):
            gdesc[j % NBUF] = pltpu.async_copy(
                half.at[src_s.at[j]], bufs[j % NBUF], gsems[j % NBUF])
        for j in range(NSLAB_W):
            b = j % NBUF
            gdesc[b].wait()
            sdesc[b] = pltpu.async_copy(
                bufs[b], acc.at[dst_s.at[j]], ssems[b], add=True)
            nxt = j + LOOK
            if nxt < NSLAB_W:
                bn = nxt % NBUF
                if sdesc[bn] is not None:
                    sdesc[bn].wait()
                gdesc[bn] = pltpu.async_copy(
                    half.at[src_s.at[nxt]], bufs[bn], gsems[bn])
        for j in range(NSLAB_W - NBUF, NSLAB_W):
            sdesc[j % NBUF].wait()

        plsc.subcore_barrier()
        _write_back(acc, out, cid, sid)

    return sc_scatter


def _sc_scatter_wide(table, src_p, dst_p):
    return _make_sc_scatter_wide()(table, src_p, dst_p)


@functools.lru_cache(maxsize=None)
def _make_sc_deg():
    @functools.partial(
        pl.kernel,
        out_type=jax.ShapeDtypeStruct((NC, N, DH), jnp.float32),
        mesh=_mesh(),
        scratch_types=[
            pltpu.VMEM((NSLAB, BATCH), jnp.int32),
            pltpu.VMEM((BATCH, DH), jnp.float32),
            pltpu.VMEM((ZROWS, DH), jnp.float32),
            pltpu.VMEM_SHARED((ACC_ROWS, DH), jnp.float32),
        ],
    )
    def sc_deg(dst_idx, out, dst_s, rowbuf, zbuf, acc):
        cid = lax.axis_index("c")
        sid = lax.axis_index("s")
        wid = cid * NS + sid
        pltpu.sync_copy(dst_idx.at[wid], dst_s)
        _const_fill(rowbuf, BATCH, DH, 1.0)
        _zero_fill(zbuf, ZROWS, DH)
        _zero_acc(acc, zbuf, ZROWS, sid)
        plsc.subcore_barrier()
        for j in range(NSLAB):
            pltpu.sync_copy(rowbuf, acc.at[dst_s.at[j]], add=True)
        plsc.subcore_barrier()
        _write_back(acc, out, cid, sid)

    return sc_deg


def _sc_deg(dst_p):
    return _make_sc_deg()(dst_p)


def _dinv_from_deg(deg_blk):
    deg = deg_blk[0, :, 0:1] + deg_blk[1, :, 0:1] + 1.0
    return lax.rsqrt(deg)


def _k1_body(feat_ref, w1_ref, deg_ref, out_ref):
    dinv = _dinv_from_deg(deg_ref[...])
    h = jnp.dot(feat_ref[...], w1_ref[...],
                preferred_element_type=jnp.float32) * dinv
    out_ref[0] = h[:, :DH]
    out_ref[1] = h[:, DH:]


def _k_combine_body(agg_ref, hs_ref, deg_ref, b_ref, w_ref, out_ref):
    dinv = _dinv_from_deg(deg_ref[...])
    s = jnp.concatenate(
        [agg_ref[0] + hs_ref[0], agg_ref[1] + hs_ref[1]], axis=-1)
    t = jnp.maximum(s * dinv + b_ref[...], 0.0)
    r = jnp.dot(t, w_ref[...], preferred_element_type=jnp.float32) * dinv
    out_ref[0] = r[:, :DH]
    out_ref[1] = r[:, DH:]


def _k_combine3_body(agg_ref, hs_ref, deg_ref, b_ref, out_ref):
    dinv = _dinv_from_deg(deg_ref[...])
    s = jnp.concatenate(
        [agg_ref[0] + hs_ref[0], agg_ref[1] + hs_ref[1]], axis=-1)
    t = jnp.maximum(s * dinv + b_ref[...], 0.0) * dinv
    out_ref[0] = t[:, :DH]
    out_ref[1] = t[:, DH:]


def _k_emb_body(agg_ref, hs_ref, deg_ref, w_ref, b_ref, out_ref):
    dinv = _dinv_from_deg(deg_ref[...])
    u = jnp.concatenate(
        [agg_ref[0] + hs_ref[0], agg_ref[1] + hs_ref[1]], axis=-1) * dinv
    out_ref[...] = jnp.dot(u, w_ref[...],
                           preferred_element_type=jnp.float32) + b_ref[...]

def _k_q_body(emb_ref, embt_ref, out_ref):
    xi = emb_ref[:, 0:1]
    yi = emb_ref[:, 1:2]
    xj = embt_ref[0:1, :]
    yj = embt_ref[1:2, :]
    dx = xi - xj
    dy = yi - yj
    dist = jnp.sqrt(dx * dx + dy * dy)
    out_ref[...] = 1.0 / (1.0 + 0.5 * dist)


BM = 1000
QBM = 200


def kernel(features, edge_index, W1, b1, W2, b2, W3, b3):
    src = edge_index[0]
    dst = edge_index[1]
    e = src.shape[0]
    src_flat = jnp.concatenate([src, jnp.zeros((EPAD - e,), jnp.int32)])
    dst_flat = jnp.concatenate(
        [dst, jnp.full((EPAD - e,), TRASH_ROW, jnp.int32)])
    src_n = src_flat.reshape(NW, NSLAB, BATCH)
    dst_n = dst_flat.reshape(NW, NSLAB, BATCH)
    src_w = src_flat[:WEPAD].reshape(NS, NSLAB_W, BATCH)
    dst_w = dst_flat[:WEPAD].reshape(NS, NSLAB_W, BATCH)
    b1r = b1.reshape(1, NHID)
    b2r = b2.reshape(1, NHID)
    b3r = b3.reshape(1, OUT_DIM)

    degpart = _sc_deg(dst_n)

    grid = N // BM
    h1s = pl.pallas_call(
        _k1_body,
        grid=(grid,),
        in_specs=[
            pl.BlockSpec((BM, IN_DIM), lambda i: (i, 0)),
            pl.BlockSpec((IN_DIM, NHID), lambda i: (0, 0)),
            pl.BlockSpec((NC, BM, DH), lambda i: (0, i, 0)),
        ],
        out_specs=pl.BlockSpec((NC, BM, DH), lambda i: (0, i, 0)),
        out_shape=jax.ShapeDtypeStruct((NC, N, DH), jnp.float32),
    )(features, W1, degpart)

    def combine(body, agg, hs, b, w, out_shape, out_spec):
        return pl.pallas_call(
            body,
            grid=(grid,),
            in_specs=[
                pl.BlockSpec((NC, BM, DH), lambda i: (0, i, 0)),
                pl.BlockSpec((NC, BM, DH), lambda i: (0, i, 0)),
                pl.BlockSpec((NC, BM, DH), lambda i: (0, i, 0)),
                pl.BlockSpec((1, NHID), lambda i: (0, 0)),
                pl.BlockSpec((NHID, w.shape[1]), lambda i: (0, 0)),
            ],
            out_specs=out_spec,
            out_shape=out_shape,
        )(agg, hs, degpart, b, w)

    wide_shape = jax.ShapeDtypeStruct((NC, N, DH), jnp.float32)
    wide_spec = pl.BlockSpec((NC, BM, DH), lambda i: (0, i, 0))
    agg1 = _sc_scatter_wide(h1s, src_w, dst_w)
    h2s = combine(_k_combine_body, agg1, h1s, b1r, W2, wide_shape, wide_spec)
    agg2 = _sc_scatter_wide(h2s, src_w, dst_w)
    t3s = pl.pallas_call(
        _k_combine3_body,
        grid=(grid,),
        in_specs=[
            pl.BlockSpec((NC, BM, DH), lambda i: (0, i, 0)),
            pl.BlockSpec((NC, BM, DH), lambda i: (0, i, 0)),
            pl.BlockSpec((NC, BM, DH), lambda i: (0, i, 0)),
            pl.BlockSpec((1, NHID), lambda i: (0, 0)),
        ],
        out_specs=wide_spec,
        out_shape=wide_shape,
    )(agg2, h2s, degpart, b2r)
    agg3 = _sc_scatter_wide(t3s, src_w, dst_w)

    emb = pl.pallas_call(
        _k_emb_body,
        grid=(grid,),
        in_specs=[
            pl.BlockSpec((NC, BM, DH), lambda i: (0, i, 0)),
            pl.BlockSpec((NC, BM, DH), lambda i: (0, i, 0)),
            pl.BlockSpec((NC, BM, DH), lambda i: (0, i, 0)),
            pl.BlockSpec((NHID, OUT_DIM), lambda i: (0, 0)),
            pl.BlockSpec((1, OUT_DIM), lambda i: (0, 0)),
        ],
        out_specs=pl.BlockSpec((BM, OUT_DIM), lambda i: (i, 0)),
        out_shape=jax.ShapeDtypeStruct((N, OUT_DIM), jnp.float32),
    )(agg3, t3s, degpart, W3, b3r)

    q = pl.pallas_call(
        _k_q_body,
        grid=(N // QBM,),
        in_specs=[
            pl.BlockSpec((QBM, OUT_DIM), lambda i: (i, 0)),
            pl.BlockSpec((OUT_DIM, N), lambda i: (0, 0)),
        ],
        out_specs=pl.BlockSpec((QBM, N), lambda i: (i, 0)),
        out_shape=jax.ShapeDtypeStruct((N, N), jnp.float32),
    )(emb, emb.T)

    return emb, q


# speedup vs baseline: 5.1647x; 1.0194x over previous
import functools

import jax
import jax.numpy as jnp
from jax import lax
from jax.experimental import pallas as pl
from jax.experimental.pallas import tpu as pltpu
from jax.experimental.pallas import tpu_sc as plsc

N = 5000
IN_DIM = 1000
NHID = 256
OUT_DIM = 2

NC = 2
NS = 16
NW = NC * NS
BATCH = 128
NSLAB = 8
EPW = BATCH * NSLAB
EPAD = EPW * NW
TRASH_ROW = N
ACC_ROWS = 5120
RPT = ACC_ROWS // NS
TAIL = N - (NS - 1) * RPT

@functools.lru_cache(maxsize=None)
def _mesh():
    return plsc.VectorSubcoreMesh(
        core_axis_name="c", subcore_axis_name="s",
        num_cores=NC, num_subcores=NS)


def _const_fill(buf, rows, d, val):
    def body(i, _):
        row = i // (d // 16)
        col = (i % (d // 16)) * 16
        buf[row, pl.ds(col, 16)] = jnp.full((16,), val, jnp.float32)
        return ()
    lax.fori_loop(0, rows * (d // 16), body, ())


def _zero_fill(zbuf, zrows, d):
    _const_fill(zbuf, zrows, d, 0.0)


def _zero_acc(acc, zbuf, zrows, sid):
    base = sid * RPT
    nfull = RPT // zrows
    rem = RPT - nfull * zrows
    for k in range(nfull):
        pltpu.sync_copy(zbuf, acc.at[pl.ds(base + k * zrows, zrows)])
    if rem:
        pltpu.sync_copy(zbuf.at[pl.ds(0, rem)],
                        acc.at[pl.ds(base + nfull * zrows, rem)])


def _write_back(acc, out, cid, sid):
    base = sid * RPT
    pltpu.sync_copy(acc.at[pl.ds(base, TAIL)],
                    out.at[cid, pl.ds(base, TAIL)])

    @pl.when(sid < NS - 1)
    def _():
        pltpu.sync_copy(acc.at[pl.ds(base + TAIL, RPT - TAIL)],
                        out.at[cid, pl.ds(base + TAIL, RPT - TAIL)])


DH = NHID // NC
BATCH_W = 64
NSLAB_W = 30
WEPAD = NSLAB_W * BATCH_W * NS
NBUF = 9
LOOK = 3
ZROWS = 16


@functools.lru_cache(maxsize=None)
def _make_sc_scatter_wide():
    @functools.partial(
        pl.kernel,
        out_type=jax.ShapeDtypeStruct((NC, N, DH), jnp.float32),
        mesh=_mesh(),
        scratch_types=[
            pltpu.VMEM((NSLAB_W, BATCH_W), jnp.int32),
            pltpu.VMEM((NSLAB_W, BATCH_W), jnp.int32),
            [pltpu.VMEM((BATCH_W, DH), jnp.float32)] * NBUF,
            pltpu.VMEM((ZROWS, DH), jnp.float32),
            pltpu.VMEM_SHARED((ACC_ROWS, DH), jnp.float32),
            [pltpu.SemaphoreType.DMA] * NBUF,
            [pltpu.SemaphoreType.DMA] * NBUF,
        ],
    )
    def sc_scatter(table, src_idx, dst_idx, out,
                   src_s, dst_s, bufs, zbuf, acc, gsems, ssems):
        cid = lax.axis_index("c")
        sid = lax.axis_index("s")
        pltpu.sync_copy(src_idx.at[sid], src_s)
        pltpu.sync_copy(dst_idx.at[sid], dst_s)
        _zero_fill(zbuf, ZROWS, DH)
        _zero_acc(acc, zbuf, ZROWS, sid)
        plsc.subcore_barrier()

        half = table.at[cid]
        gdesc = [None] * NBUF
        sdesc = [None] * NBUF
        for j in range(LOOK):
            gdesc[j % NBUF] = pltpu.async_copy(
                half.at[src_s.at[j]], bufs[j % NBUF], gsems[j % NBUF])
        for j in range(NSLAB_W):
            b = j % NBUF
            gdesc[b].wait()
            sdesc[b] = pltpu.async_copy(
                bufs[b], acc.at[dst_s.at[j]], ssems[b], add=True)
            nxt = j + LOOK
            if nxt < NSLAB_W:
                bn = nxt % NBUF
                if sdesc[bn] is not None:
                    sdesc[bn].wait()
                gdesc[bn] = pltpu.async_copy(
                    half.at[src_s.at[nxt]], bufs[bn], gsems[bn])
        for j in range(NSLAB_W - NBUF, NSLAB_W):
            sdesc[j % NBUF].wait()

        plsc.subcore_barrier()
        _write_back(acc, out, cid, sid)

    return sc_scatter


def _sc_scatter_wide(table, src_p, dst_p):
    return _make_sc_scatter_wide()(table, src_p, dst_p)


@functools.lru_cache(maxsize=None)
def _make_sc_deg():
    @functools.partial(
        pl.kernel,
        out_type=jax.ShapeDtypeStruct((NC, N, DH), jnp.float32),
        mesh=_mesh(),
        scratch_types=[
            pltpu.VMEM((NSLAB, BATCH), jnp.int32),
            pltpu.VMEM((BATCH, DH), jnp.float32),
            pltpu.VMEM((ZROWS, DH), jnp.float32),
            pltpu.VMEM_SHARED((ACC_ROWS, DH), jnp.float32),
        ],
    )
    def sc_deg(dst_idx, out, dst_s, rowbuf, zbuf, acc):
        cid = lax.axis_index("c")
        sid = lax.axis_index("s")
        wid = cid * NS + sid
        pltpu.sync_copy(dst_idx.at[wid], dst_s)
        _const_fill(rowbuf, BATCH, DH, 1.0)
        _zero_fill(zbuf, ZROWS, DH)
        _zero_acc(acc, zbuf, ZROWS, sid)
        plsc.subcore_barrier()
        for j in range(NSLAB):
            pltpu.sync_copy(rowbuf, acc.at[dst_s.at[j]], add=True)
        plsc.subcore_barrier()
        _write_back(acc, out, cid, sid)

    return sc_deg


def _sc_deg(dst_p):
    return _make_sc_deg()(dst_p)


def _dinv_from_deg(deg_blk):
    deg = deg_blk[0, :, 0:1] + deg_blk[1, :, 0:1] + 1.0
    return lax.rsqrt(deg)


def _k1a_body(feat_ref, w1_ref, out_ref):
    out_ref[...] = jnp.dot(feat_ref[...], w1_ref[...],
                           preferred_element_type=jnp.float32)


def _k1b_body(h_ref, deg_ref, out_ref):
    dinv = _dinv_from_deg(deg_ref[...])
    h = h_ref[...] * dinv
    out_ref[0] = h[:, :DH]
    out_ref[1] = h[:, DH:]


def _k_combine_body(agg_ref, hs_ref, deg_ref, b_ref, w_ref, out_ref):
    dinv = _dinv_from_deg(deg_ref[...])
    s = jnp.concatenate(
        [agg_ref[0] + hs_ref[0], agg_ref[1] + hs_ref[1]], axis=-1)
    t = jnp.maximum(s * dinv + b_ref[...], 0.0)
    r = jnp.dot(t, w_ref[...], preferred_element_type=jnp.float32) * dinv
    out_ref[0] = r[:, :DH]
    out_ref[1] = r[:, DH:]


def _k_combine3_body(agg_ref, hs_ref, deg_ref, b_ref, out_ref):
    dinv = _dinv_from_deg(deg_ref[...])
    s = jnp.concatenate(
        [agg_ref[0] + hs_ref[0], agg_ref[1] + hs_ref[1]], axis=-1)
    t = jnp.maximum(s * dinv + b_ref[...], 0.0) * dinv
    out_ref[0] = t[:, :DH]
    out_ref[1] = t[:, DH:]


def _k_emb_body(agg_ref, hs_ref, deg_ref, w_ref, b_ref, out_ref):
    dinv = _dinv_from_deg(deg_ref[...])
    u = jnp.concatenate(
        [agg_ref[0] + hs_ref[0], agg_ref[1] + hs_ref[1]], axis=-1) * dinv
    out_ref[...] = jnp.dot(u, w_ref[...],
                           preferred_element_type=jnp.float32) + b_ref[...]

def _k_q_body(emb_ref, embt_ref, out_ref):
    xi = emb_ref[:, 0:1]
    yi = emb_ref[:, 1:2]
    xj = embt_ref[0:1, :]
    yj = embt_ref[1:2, :]
    dx = xi - xj
    dy = yi - yj
    dist = jnp.sqrt(dx * dx + dy * dy)
    out_ref[...] = 1.0 / (1.0 + 0.5 * dist)


BM = 1000
QBM = 200


def kernel(features, edge_index, W1, b1, W2, b2, W3, b3):
    src = edge_index[0]
    dst = edge_index[1]
    e = src.shape[0]
    src_flat = jnp.concatenate([src, jnp.zeros((EPAD - e,), jnp.int32)])
    dst_flat = jnp.concatenate(
        [dst, jnp.full((EPAD - e,), TRASH_ROW, jnp.int32)])
    src_n = src_flat.reshape(NW, NSLAB, BATCH)
    dst_n = dst_flat.reshape(NW, NSLAB, BATCH)
    src_w = src_flat[:WEPAD].reshape(NS, NSLAB_W, BATCH_W)
    dst_w = dst_flat[:WEPAD].reshape(NS, NSLAB_W, BATCH_W)
    b1r = b1.reshape(1, NHID)
    b2r = b2.reshape(1, NHID)
    b3r = b3.reshape(1, OUT_DIM)

    grid = N // BM
    degpart = _sc_deg(dst_n)
    h1p = pl.pallas_call(
        _k1a_body,
        grid=(grid,),
        in_specs=[
            pl.BlockSpec((BM, IN_DIM), lambda i: (i, 0)),
            pl.BlockSpec((IN_DIM, NHID), lambda i: (0, 0)),
        ],
        out_specs=pl.BlockSpec((BM, NHID), lambda i: (i, 0)),
        out_shape=jax.ShapeDtypeStruct((N, NHID), jnp.float32),
    )(features, W1)
    h1s = pl.pallas_call(
        _k1b_body,
        grid=(grid,),
        in_specs=[
            pl.BlockSpec((BM, NHID), lambda i: (i, 0)),
            pl.BlockSpec((NC, BM, DH), lambda i: (0, i, 0)),
        ],
        out_specs=pl.BlockSpec((NC, BM, DH), lambda i: (0, i, 0)),
        out_shape=jax.ShapeDtypeStruct((NC, N, DH), jnp.float32),
    )(h1p, degpart)

    def combine(body, agg, hs, b, w, out_shape, out_spec):
        return pl.pallas_call(
            body,
            grid=(grid,),
            in_specs=[
                pl.BlockSpec((NC, BM, DH), lambda i: (0, i, 0)),
                pl.BlockSpec((NC, BM, DH), lambda i: (0, i, 0)),
                pl.BlockSpec((NC, BM, DH), lambda i: (0, i, 0)),
                pl.BlockSpec((1, NHID), lambda i: (0, 0)),
                pl.BlockSpec((NHID, w.shape[1]), lambda i: (0, 0)),
            ],
            out_specs=out_spec,
            out_shape=out_shape,
        )(agg, hs, degpart, b, w)

    wide_shape = jax.ShapeDtypeStruct((NC, N, DH), jnp.float32)
    wide_spec = pl.BlockSpec((NC, BM, DH), lambda i: (0, i, 0))
    agg1 = _sc_scatter_wide(h1s, src_w, dst_w)
    h2s = combine(_k_combine_body, agg1, h1s, b1r, W2, wide_shape, wide_spec)
    agg2 = _sc_scatter_wide(h2s, src_w, dst_w)
    t3s = pl.pallas_call(
        _k_combine3_body,
        grid=(grid,),
        in_specs=[
            pl.BlockSpec((NC, BM, DH), lambda i: (0, i, 0)),
            pl.BlockSpec((NC, BM, DH), lambda i: (0, i, 0)),
            pl.BlockSpec((NC, BM, DH), lambda i: (0, i, 0)),
            pl.BlockSpec((1, NHID), lambda i: (0, 0)),
        ],
        out_specs=wide_spec,
        out_shape=wide_shape,
    )(agg2, h2s, degpart, b2r)
    agg3 = _sc_scatter_wide(t3s, src_w, dst_w)

    emb = pl.pallas_call(
        _k_emb_body,
        grid=(grid,),
        in_specs=[
            pl.BlockSpec((NC, BM, DH), lambda i: (0, i, 0)),
            pl.BlockSpec((NC, BM, DH), lambda i: (0, i, 0)),
            pl.BlockSpec((NC, BM, DH), lambda i: (0, i, 0)),
            pl.BlockSpec((NHID, OUT_DIM), lambda i: (0, 0)),
            pl.BlockSpec((1, OUT_DIM), lambda i: (0, 0)),
        ],
        out_specs=pl.BlockSpec((BM, OUT_DIM), lambda i: (i, 0)),
        out_shape=jax.ShapeDtypeStruct((N, OUT_DIM), jnp.float32),
    )(agg3, t3s, degpart, W3, b3r)

    q = pl.pallas_call(
        _k_q_body,
        grid=(N // QBM,),
        in_specs=[
            pl.BlockSpec((QBM, OUT_DIM), lambda i: (i, 0)),
            pl.BlockSpec((OUT_DIM, N), lambda i: (0, 0)),
        ],
        out_specs=pl.BlockSpec((QBM, N), lambda i: (i, 0)),
        out_shape=jax.ShapeDtypeStruct((N, N), jnp.float32),
    )(emb, emb.T)

    return emb, q
